# baseline (speedup 1.0000x reference)
import jax
import jax.numpy as jnp
from jax import lax
from jax.experimental import pallas as pl

DEPTH = 1000
BATCH = 1024
GROUP = 26
BLOCK_D = 200


def _onehot_body(idxt_ref, out_ref):
    idxt = idxt_ref[...]
    d0 = pl.program_id(0) * BLOCK_D
    dio = d0 + lax.broadcasted_iota(jnp.int32, (GROUP, BLOCK_D, BATCH), 1)
    out_ref[...] = (idxt[:, None, :] == dio).astype(jnp.float32)


def kernel(inputs):
    idxt = inputs.T
    out_t = pl.pallas_call(
        _onehot_body,
        grid=(DEPTH // BLOCK_D,),
        in_specs=[pl.BlockSpec((GROUP, BATCH), lambda i: (0, 0))],
        out_specs=pl.BlockSpec((GROUP, BLOCK_D, BATCH), lambda i: (0, i, 0)),
        out_shape=jax.ShapeDtypeStruct((GROUP, DEPTH, BATCH), jnp.float32),
    )(idxt)
    return out_t.transpose(2, 0, 1)
